# baseline (speedup 1.0000x reference)
import functools

import jax
import jax.numpy as jnp
from jax import lax
from jax.experimental import pallas as pl
from jax.experimental.pallas import tpu as pltpu
from jax.experimental.pallas import tpu_sc as plsc

NC, NS, L = 2, 16, 16
NW = NC * NS
B, S, D = 256, 16, 128
BPW = B // NW
HALF = S * (2 * S + 2)
RPB = 2 * HALF
NCH = 17


def _body(nid_hbm, rot_hbm, sin_hbm, sout_hbm, win_hbm, wout_hbm,
          ofeat_hbm, owt_hbm,
          nid_v, n1in, n1out, w1in, w1out, n2in, n2out, w2in, w2out,
          idx_v, pref_v, wout_v, outbuf, colb, sem):
    wid = lax.axis_index("s") * NC + lax.axis_index("c")
    base_b = wid * BPW
    pltpu.sync_copy(nid_hbm.at[pl.ds(base_b, BPW)], nid_v)
    l1 = [pltpu.async_copy(sin_hbm.at[nid_v], n1in, sem),
          pltpu.async_copy(sout_hbm.at[nid_v], n1out, sem),
          pltpu.async_copy(win_hbm.at[nid_v], w1in, sem),
          pltpu.async_copy(wout_hbm.at[nid_v], w1out, sem)]
    for d in l1:
        d.wait()

    iota = lax.iota(jnp.int32, L)
    zeros = jnp.zeros((L,), jnp.int32)

    for d1 in range(2):
        n1ref = n1in if d1 == 0 else n1out
        w1ref = w1in if d1 == 0 else w1out
        roff = d1 * HALF

        @pl.loop(0, BPW)
        def _half(i):
            ivec = zeros + i
            l2 = [pltpu.async_copy(sin_hbm.at[n1ref.at[i]], n2in, sem),
                  pltpu.async_copy(sout_hbm.at[n1ref.at[i]], n2out, sem),
                  pltpu.async_copy(win_hbm.at[n1ref.at[i]], w2in, sem),
                  pltpu.async_copy(wout_hbm.at[n1ref.at[i]], w2out, sem)]
            n1row = plsc.load_gather(n1ref, [ivec, iota])
            w1row = plsc.load_gather(w1ref, [ivec, iota])
            pos0 = iota * 34
            pos17 = pos0 + 17
            plsc.store_scatter(idx_v, [pos0 >> 5, pos0 & 31], n1row)
            plsc.store_scatter(idx_v, [pos17 >> 5, pos17 & 31], n1row)
            plsc.store_scatter(pref_v, [pos0], w1row)
            plsc.store_scatter(pref_v, [pos17], w1row)
            w1sq = w1row * w1row
            plsc.store_scatter(wout_v, [pos0], w1sq)
            plsc.store_scatter(wout_v, [pos17], w1sq)
            for d in l2:
                d.wait()
            for j in range(S):
                pin = j * 34 + 1 + iota
                pout = j * 34 + 18 + iota
                plsc.store_scatter(idx_v, [pin >> 5, pin & 31], n2in[j, :])
                plsc.store_scatter(idx_v, [pout >> 5, pout & 31], n2out[j, :])
            fg = [pltpu.async_copy(rot_hbm.at[idx_v.at[c]],
                                   outbuf.at[pl.ds(c * 32, 32), :], sem)
                  for c in range(NCH)]
            for j in range(S):
                w1j = plsc.load_gather(
                    w1ref, [ivec, jnp.full((L,), j, jnp.int32)])
                pin = j * 34 + 1 + iota
                pout = j * 34 + 18 + iota
                w2i = w2in[j, :]
                w2o = w2out[j, :]
                plsc.store_scatter(pref_v, [pin], w2i)
                plsc.store_scatter(pref_v, [pout], w2o)
                plsc.store_scatter(wout_v, [pin], w2i * w1j)
                plsc.store_scatter(wout_v, [pout], w2o * w1j)
            for d in fg:
                d.wait()
            for c in range(HALF // L):
                rows = c * L + iota
                f127 = plsc.load_gather(outbuf, [rows, zeros])
                plsc.store_scatter(colb, [rows, zeros], f127)
                plsc.store_scatter(outbuf, [rows, zeros],
                                   pref_v[pl.ds(c * L, L)])
            pltpu.sync_copy(outbuf,
                            ofeat_hbm.at[base_b + i, pl.ds(roff, HALF),
                                         pl.ds(0, D)])
            pltpu.sync_copy(colb,
                            ofeat_hbm.at[base_b + i, pl.ds(roff, HALF),
                                         pl.ds(D, 1)])
            pltpu.sync_copy(wout_v,
                            owt_hbm.at[base_b + i, pl.ds(roff, HALF)])


def kernel(node_ids, features, in_sample, out_sample, in_sample_amnt,
           out_sample_amnt):
    rot = jnp.concatenate(
        [features[:, D - 1:D], features[:, :D - 1]], axis=1)
    w_in = jnp.reshape(in_sample_amnt[:, :S, :1], (-1, S))
    w_out = jnp.reshape(out_sample_amnt[:, :S, :1], (-1, S))
    sin = in_sample[:, :S].astype(jnp.int32)
    sout = out_sample[:, :S].astype(jnp.int32)
    nid = node_ids.astype(jnp.int32)

    mesh = plsc.VectorSubcoreMesh(core_axis_name="c", subcore_axis_name="s")
    run = functools.partial(
        pl.kernel,
        out_type=(jax.ShapeDtypeStruct((B, RPB, D + 1), jnp.float32),
                  jax.ShapeDtypeStruct((B, RPB), jnp.float32)),
        mesh=mesh,
        compiler_params=pltpu.CompilerParams(
            use_tc_tiling_on_sc=False, needs_layout_passes=False),
        scratch_types=[
            pltpu.VMEM((BPW,), jnp.int32),
            pltpu.VMEM((BPW, S), jnp.int32),
            pltpu.VMEM((BPW, S), jnp.int32),
            pltpu.VMEM((BPW, S), jnp.float32),
            pltpu.VMEM((BPW, S), jnp.float32),
            pltpu.VMEM((S, S), jnp.int32),
            pltpu.VMEM((S, S), jnp.int32),
            pltpu.VMEM((S, S), jnp.float32),
            pltpu.VMEM((S, S), jnp.float32),
            pltpu.VMEM((NCH, 32), jnp.int32),
            pltpu.VMEM((HALF,), jnp.float32),
            pltpu.VMEM((HALF,), jnp.float32),
            pltpu.VMEM((HALF, D), jnp.float32),
            pltpu.VMEM((HALF, 1), jnp.float32),
            pltpu.SemaphoreType.DMA,
        ],
    )(_body)
    return run(nid, rot, sin, sout, w_in, w_out)


# speedup vs baseline: 1.0363x; 1.0363x over previous
import functools

import jax
import jax.numpy as jnp
from jax import lax
from jax.experimental import pallas as pl
from jax.experimental.pallas import tpu as pltpu
from jax.experimental.pallas import tpu_sc as plsc

NC, NS, L = 2, 16, 16
NW = NC * NS
B, S, D = 256, 16, 128
BPW = B // NW
HALF = S * (2 * S + 2)
RPB = 2 * HALF
NCH = 17


def _rot_body(f_ref, o_ref):
    o_ref[...] = pltpu.roll(f_ref[...], 1, 1)


def _body(nid_hbm, rot_hbm, sin_hbm, sout_hbm, win_hbm, wout_hbm,
          ofeat_hbm, owt_hbm,
          nid_v, n1in, n1out, w1in, w1out, n2in, n2out, w2in, w2out,
          idx_v, pref_v, wout_v, outbuf, colb, sem):
    wid = lax.axis_index("s") * NC + lax.axis_index("c")
    base_b = wid * BPW
    pltpu.sync_copy(nid_hbm.at[pl.ds(base_b, BPW)], nid_v)
    l1 = [pltpu.async_copy(sin_hbm.at[nid_v], n1in, sem),
          pltpu.async_copy(sout_hbm.at[nid_v], n1out, sem),
          pltpu.async_copy(win_hbm.at[nid_v], w1in, sem),
          pltpu.async_copy(wout_hbm.at[nid_v], w1out, sem)]
    for d in l1:
        d.wait()

    iota = lax.iota(jnp.int32, L)
    zeros = jnp.zeros((L,), jnp.int32)

    for d1 in range(2):
        n1ref = n1in if d1 == 0 else n1out
        w1ref = w1in if d1 == 0 else w1out
        roff = d1 * HALF

        @pl.loop(0, BPW)
        def _half(i):
            ivec = zeros + i
            l2 = [pltpu.async_copy(sin_hbm.at[n1ref.at[i]], n2in, sem),
                  pltpu.async_copy(sout_hbm.at[n1ref.at[i]], n2out, sem),
                  pltpu.async_copy(win_hbm.at[n1ref.at[i]], w2in, sem),
                  pltpu.async_copy(wout_hbm.at[n1ref.at[i]], w2out, sem)]
            n1row = plsc.load_gather(n1ref, [ivec, iota])
            w1row = plsc.load_gather(w1ref, [ivec, iota])
            pos0 = iota * 34
            pos17 = pos0 + 17
            plsc.store_scatter(idx_v, [pos0 >> 5, pos0 & 31], n1row)
            plsc.store_scatter(idx_v, [pos17 >> 5, pos17 & 31], n1row)
            plsc.store_scatter(pref_v, [pos0], w1row)
            plsc.store_scatter(pref_v, [pos17], w1row)
            w1sq = w1row * w1row
            plsc.store_scatter(wout_v, [pos0], w1sq)
            plsc.store_scatter(wout_v, [pos17], w1sq)
            for d in l2:
                d.wait()
            for j in range(S):
                pin = j * 34 + 1 + iota
                pout = j * 34 + 18 + iota
                plsc.store_scatter(idx_v, [pin >> 5, pin & 31], n2in[j, :])
                plsc.store_scatter(idx_v, [pout >> 5, pout & 31], n2out[j, :])
            fg = [pltpu.async_copy(rot_hbm.at[idx_v.at[c]],
                                   outbuf.at[pl.ds(c * 32, 32), :], sem)
                  for c in range(NCH)]
            for j in range(S):
                w1j = plsc.load_gather(
                    w1ref, [ivec, jnp.full((L,), j, jnp.int32)])
                pin = j * 34 + 1 + iota
                pout = j * 34 + 18 + iota
                w2i = w2in[j, :]
                w2o = w2out[j, :]
                plsc.store_scatter(pref_v, [pin], w2i)
                plsc.store_scatter(pref_v, [pout], w2o)
                plsc.store_scatter(wout_v, [pin], w2i * w1j)
                plsc.store_scatter(wout_v, [pout], w2o * w1j)
            for d in fg:
                d.wait()
            for c in range(HALF // L):
                rows = c * L + iota
                f127 = plsc.load_gather(outbuf, [rows, zeros])
                plsc.store_scatter(colb, [rows, zeros], f127)
                plsc.store_scatter(outbuf, [rows, zeros],
                                   pref_v[pl.ds(c * L, L)])
            pltpu.sync_copy(outbuf,
                            ofeat_hbm.at[base_b + i, pl.ds(roff, HALF),
                                         pl.ds(0, D)])
            pltpu.sync_copy(colb,
                            ofeat_hbm.at[base_b + i, pl.ds(roff, HALF),
                                         pl.ds(D, 1)])
            pltpu.sync_copy(wout_v,
                            owt_hbm.at[base_b + i, pl.ds(roff, HALF)])


def kernel(node_ids, features, in_sample, out_sample, in_sample_amnt,
           out_sample_amnt):
    n = features.shape[0]
    rb = 2000
    rot = pl.pallas_call(
        _rot_body,
        grid=(n // rb,),
        in_specs=[pl.BlockSpec((rb, D), lambda i: (i, 0))],
        out_specs=pl.BlockSpec((rb, D), lambda i: (i, 0)),
        out_shape=jax.ShapeDtypeStruct((n, D), jnp.float32),
    )(features)

    mesh = plsc.VectorSubcoreMesh(core_axis_name="c", subcore_axis_name="s")
    run = functools.partial(
        pl.kernel,
        out_type=(jax.ShapeDtypeStruct((B, RPB, D + 1), jnp.float32),
                  jax.ShapeDtypeStruct((B, RPB), jnp.float32)),
        mesh=mesh,
        compiler_params=pltpu.CompilerParams(
            use_tc_tiling_on_sc=False, needs_layout_passes=False),
        scratch_types=[
            pltpu.VMEM((BPW,), jnp.int32),
            pltpu.VMEM((BPW, S), jnp.int32),
            pltpu.VMEM((BPW, S), jnp.int32),
            pltpu.VMEM((BPW, S), jnp.float32),
            pltpu.VMEM((BPW, S), jnp.float32),
            pltpu.VMEM((S, S), jnp.int32),
            pltpu.VMEM((S, S), jnp.int32),
            pltpu.VMEM((S, S), jnp.float32),
            pltpu.VMEM((S, S), jnp.float32),
            pltpu.VMEM((NCH, 32), jnp.int32),
            pltpu.VMEM((HALF,), jnp.float32),
            pltpu.VMEM((HALF,), jnp.float32),
            pltpu.VMEM((HALF, D), jnp.float32),
            pltpu.VMEM((HALF, 1), jnp.float32),
            pltpu.SemaphoreType.DMA,
        ],
    )(_body)
    return run(node_ids.astype(jnp.int32), rot,
               in_sample.astype(jnp.int32), out_sample.astype(jnp.int32),
               jnp.reshape(in_sample_amnt, (n, S)),
               jnp.reshape(out_sample_amnt, (n, S)))


# speedup vs baseline: 1.5722x; 1.5171x over previous
import functools

import jax
import jax.numpy as jnp
from jax import lax
from jax.experimental import pallas as pl
from jax.experimental.pallas import tpu as pltpu
from jax.experimental.pallas import tpu_sc as plsc

NC, NS, L = 2, 16, 16
NW = NC * NS
B, S, D = 256, 16, 128
BPW = B // NW
HALF = S * (2 * S + 2)
RPB = 2 * HALF
NCH = 17


def _sc_body(nid_hbm, feat_hbm, sin_hbm, sout_hbm, win_hbm, wout_hbm,
             fa_hbm, pf_hbm, wt_hbm,
             nid_v, n1in, n1out, w1in, w1out, n2in, n2out, w2in, w2out,
             idx_v, pref8, wout8, outbuf, sem):
    wid = lax.axis_index("s") * NC + lax.axis_index("c")
    base_b = wid * BPW
    pltpu.sync_copy(nid_hbm.at[pl.ds(base_b, BPW)], nid_v)
    l1 = [pltpu.async_copy(sin_hbm.at[nid_v], n1in, sem),
          pltpu.async_copy(sout_hbm.at[nid_v], n1out, sem),
          pltpu.async_copy(win_hbm.at[nid_v], w1in, sem),
          pltpu.async_copy(wout_hbm.at[nid_v], w1out, sem)]
    for d in l1:
        d.wait()

    iota = lax.iota(jnp.int32, L)
    zeros = jnp.zeros((L,), jnp.int32)

    for d1 in range(2):
        n1ref = n1in if d1 == 0 else n1out
        w1ref = w1in if d1 == 0 else w1out
        roff = d1 * HALF

        @pl.loop(0, BPW)
        def _half(i):
            ivec = zeros + i
            l2 = [pltpu.async_copy(sin_hbm.at[n1ref.at[i]], n2in, sem),
                  pltpu.async_copy(sout_hbm.at[n1ref.at[i]], n2out, sem),
                  pltpu.async_copy(win_hbm.at[n1ref.at[i]], w2in, sem),
                  pltpu.async_copy(wout_hbm.at[n1ref.at[i]], w2out, sem)]
            n1row = plsc.load_gather(n1ref, [ivec, iota])
            w1row = plsc.load_gather(w1ref, [ivec, iota])
            pos0 = iota * 34
            pos17 = pos0 + 17
            plsc.store_scatter(idx_v, [pos0 >> 5, pos0 & 31], n1row)
            plsc.store_scatter(idx_v, [pos17 >> 5, pos17 & 31], n1row)
            plsc.store_scatter(pref8, [pos0, ivec], w1row)
            plsc.store_scatter(pref8, [pos17, ivec], w1row)
            w1sq = w1row * w1row
            plsc.store_scatter(wout8, [pos0, ivec], w1sq)
            plsc.store_scatter(wout8, [pos17, ivec], w1sq)
            for d in l2:
                d.wait()
            for j in range(S):
                pin = j * 34 + 1 + iota
                pout = j * 34 + 18 + iota
                plsc.store_scatter(idx_v, [pin >> 5, pin & 31], n2in[j, :])
                plsc.store_scatter(idx_v, [pout >> 5, pout & 31], n2out[j, :])
            fg = [pltpu.async_copy(feat_hbm.at[idx_v.at[c]],
                                   outbuf.at[pl.ds(c * 32, 32), :], sem)
                  for c in range(NCH)]
            for j in range(S):
                w1j = plsc.load_gather(
                    w1ref, [ivec, jnp.full((L,), j, jnp.int32)])
                pin = j * 34 + 1 + iota
                pout = j * 34 + 18 + iota
                w2i = w2in[j, :]
                w2o = w2out[j, :]
                plsc.store_scatter(pref8, [pin, ivec], w2i)
                plsc.store_scatter(pref8, [pout, ivec], w2o)
                plsc.store_scatter(wout8, [pin, ivec], w2i * w1j)
                plsc.store_scatter(wout8, [pout, ivec], w2o * w1j)
            for d in fg:
                d.wait()
            pltpu.sync_copy(outbuf,
                            fa_hbm.at[base_b + i, pl.ds(roff, HALF), :])

        pltpu.sync_copy(pref8, pf_hbm.at[wid, pl.ds(roff, HALF), :])
        pltpu.sync_copy(wout8, wt_hbm.at[wid, pl.ds(roff, HALF), :])


def _tc_body(a_ref, b_ref, o_ref):
    o_ref[0, :, :] = b_ref[...]
    for r in range(8):
        o_ref[pl.ds(1, D), r, :] = jnp.transpose(a_ref[:, r, :], (1, 0))


def kernel(node_ids, features, in_sample, out_sample, in_sample_amnt,
           out_sample_amnt):
    n = features.shape[0]
    mesh = plsc.VectorSubcoreMesh(core_axis_name="c", subcore_axis_name="s")
    run = functools.partial(
        pl.kernel,
        out_type=(jax.ShapeDtypeStruct((B, RPB, D), jnp.float32),
                  jax.ShapeDtypeStruct((NW, RPB, BPW), jnp.float32),
                  jax.ShapeDtypeStruct((NW, RPB, BPW), jnp.float32)),
        mesh=mesh,
        compiler_params=pltpu.CompilerParams(
            use_tc_tiling_on_sc=False, needs_layout_passes=False),
        scratch_types=[
            pltpu.VMEM((BPW,), jnp.int32),
            pltpu.VMEM((BPW, S), jnp.int32),
            pltpu.VMEM((BPW, S), jnp.int32),
            pltpu.VMEM((BPW, S), jnp.float32),
            pltpu.VMEM((BPW, S), jnp.float32),
            pltpu.VMEM((S, S), jnp.int32),
            pltpu.VMEM((S, S), jnp.int32),
            pltpu.VMEM((S, S), jnp.float32),
            pltpu.VMEM((S, S), jnp.float32),
            pltpu.VMEM((NCH, 32), jnp.int32),
            pltpu.VMEM((HALF, BPW), jnp.float32),
            pltpu.VMEM((HALF, BPW), jnp.float32),
            pltpu.VMEM((HALF, D), jnp.float32),
            pltpu.SemaphoreType.DMA,
        ],
    )(_sc_body)
    feat_a, pref_w, wout_w = run(
        node_ids.astype(jnp.int32), features,
        in_sample.astype(jnp.int32), out_sample.astype(jnp.int32),
        jnp.reshape(in_sample_amnt, (n, S)),
        jnp.reshape(out_sample_amnt, (n, S)))

    pref_t = jnp.reshape(jnp.transpose(pref_w, (1, 0, 2)), (RPB, B))
    wout_t = jnp.reshape(jnp.transpose(wout_w, (1, 0, 2)), (RPB, B))

    feat_t = pl.pallas_call(
        _tc_body,
        grid=(RPB // 8, B // D),
        in_specs=[pl.BlockSpec((D, 8, D), lambda r, b: (b, r, 0)),
                  pl.BlockSpec((8, D), lambda r, b: (r, b))],
        out_specs=pl.BlockSpec((D + 1, 8, D), lambda r, b: (0, r, b)),
        out_shape=jax.ShapeDtypeStruct((D + 1, RPB, B), jnp.float32),
    )(feat_a, pref_t)

    feat = jnp.transpose(feat_t, (2, 1, 0))
    weight = jnp.transpose(wout_t, (1, 0))
    return feat, weight


# speedup vs baseline: 1.6009x; 1.0183x over previous
import functools

import jax
import jax.numpy as jnp
from jax import lax
from jax.experimental import pallas as pl
from jax.experimental.pallas import tpu as pltpu
from jax.experimental.pallas import tpu_sc as plsc

NC, NS, L = 2, 16, 16
NW = NC * NS
B, S, D = 256, 16, 128
BPW = B // NW
HALF = S * (2 * S + 2)
RPB = 2 * HALF
NCH = 17


def _sc_body(nid_hbm, feat_hbm, sin_hbm, sout_hbm, win_hbm, wout_hbm,
             fa_hbm, pf_hbm, wt_hbm,
             nid_v, n1in, n1out, w1in, w1out, n2in, n2out, w2in, w2out,
             idx_v, pref8, wout8, outbuf, sem):
    wid = lax.axis_index("s") * NC + lax.axis_index("c")
    base_b = wid * BPW
    pltpu.sync_copy(nid_hbm.at[pl.ds(base_b, BPW)], nid_v)
    l1 = [pltpu.async_copy(sin_hbm.at[nid_v], n1in, sem),
          pltpu.async_copy(sout_hbm.at[nid_v], n1out, sem),
          pltpu.async_copy(win_hbm.at[nid_v], w1in, sem),
          pltpu.async_copy(wout_hbm.at[nid_v], w1out, sem)]
    for d in l1:
        d.wait()

    iota = lax.iota(jnp.int32, L)
    zeros = jnp.zeros((L,), jnp.int32)

    for d1 in range(2):
        n1ref = n1in if d1 == 0 else n1out
        w1ref = w1in if d1 == 0 else w1out
        roff = d1 * HALF

        @pl.loop(0, BPW)
        def _half(i):
            ivec = zeros + i
            l2 = [pltpu.async_copy(sin_hbm.at[n1ref.at[i]], n2in, sem),
                  pltpu.async_copy(sout_hbm.at[n1ref.at[i]], n2out, sem),
                  pltpu.async_copy(win_hbm.at[n1ref.at[i]], w2in, sem),
                  pltpu.async_copy(wout_hbm.at[n1ref.at[i]], w2out, sem)]
            n1row = plsc.load_gather(n1ref, [ivec, iota])
            w1row = plsc.load_gather(w1ref, [ivec, iota])
            pos0 = iota * 34
            pos17 = pos0 + 17
            plsc.store_scatter(idx_v, [pos0 >> 5, pos0 & 31], n1row)
            plsc.store_scatter(idx_v, [pos17 >> 5, pos17 & 31], n1row)
            plsc.store_scatter(pref8, [pos0, ivec], w1row)
            plsc.store_scatter(pref8, [pos17, ivec], w1row)
            w1sq = w1row * w1row
            plsc.store_scatter(wout8, [pos0, ivec], w1sq)
            plsc.store_scatter(wout8, [pos17, ivec], w1sq)
            for d in l2:
                d.wait()
            for j in range(S):
                pin = j * 34 + 1 + iota
                pout = j * 34 + 18 + iota
                plsc.store_scatter(idx_v, [pin >> 5, pin & 31], n2in[j, :])
                plsc.store_scatter(idx_v, [pout >> 5, pout & 31], n2out[j, :])
            fg = [pltpu.async_copy(feat_hbm.at[idx_v.at[c]],
                                   outbuf.at[pl.ds(c * 32, 32), :], sem)
                  for c in range(NCH)]
            for j in range(S):
                w1j = plsc.load_gather(
                    w1ref, [ivec, jnp.full((L,), j, jnp.int32)])
                pin = j * 34 + 1 + iota
                pout = j * 34 + 18 + iota
                w2i = w2in[j, :]
                w2o = w2out[j, :]
                plsc.store_scatter(pref8, [pin, ivec], w2i)
                plsc.store_scatter(pref8, [pout, ivec], w2o)
                plsc.store_scatter(wout8, [pin, ivec], w2i * w1j)
                plsc.store_scatter(wout8, [pout, ivec], w2o * w1j)
            for d in fg[:9]:
                d.wait()
            CUT = 9 * 32
            o1 = pltpu.async_copy(
                outbuf.at[pl.ds(0, CUT), :],
                fa_hbm.at[base_b + i, pl.ds(roff, CUT), :], sem)
            for d in fg[9:]:
                d.wait()
            o2 = pltpu.async_copy(
                outbuf.at[pl.ds(CUT, HALF - CUT), :],
                fa_hbm.at[base_b + i, pl.ds(roff + CUT, HALF - CUT), :], sem)
            o1.wait()
            o2.wait()

        pltpu.sync_copy(pref8, pf_hbm.at[wid, pl.ds(roff, HALF), :])
        pltpu.sync_copy(wout8, wt_hbm.at[wid, pl.ds(roff, HALF), :])


def _tc_body(a_ref, b_ref, o_ref):
    o_ref[0, :, :] = b_ref[...]
    for r in range(8):
        o_ref[pl.ds(1, D), r, :] = jnp.transpose(a_ref[:, r, :], (1, 0))


def kernel(node_ids, features, in_sample, out_sample, in_sample_amnt,
           out_sample_amnt):
    n = features.shape[0]
    mesh = plsc.VectorSubcoreMesh(core_axis_name="c", subcore_axis_name="s")
    run = functools.partial(
        pl.kernel,
        out_type=(jax.ShapeDtypeStruct((B, RPB, D), jnp.float32),
                  jax.ShapeDtypeStruct((NW, RPB, BPW), jnp.float32),
                  jax.ShapeDtypeStruct((NW, RPB, BPW), jnp.float32)),
        mesh=mesh,
        compiler_params=pltpu.CompilerParams(
            use_tc_tiling_on_sc=False, needs_layout_passes=False),
        scratch_types=[
            pltpu.VMEM((BPW,), jnp.int32),
            pltpu.VMEM((BPW, S), jnp.int32),
            pltpu.VMEM((BPW, S), jnp.int32),
            pltpu.VMEM((BPW, S), jnp.float32),
            pltpu.VMEM((BPW, S), jnp.float32),
            pltpu.VMEM((S, S), jnp.int32),
            pltpu.VMEM((S, S), jnp.int32),
            pltpu.VMEM((S, S), jnp.float32),
            pltpu.VMEM((S, S), jnp.float32),
            pltpu.VMEM((NCH, 32), jnp.int32),
            pltpu.VMEM((HALF, BPW), jnp.float32),
            pltpu.VMEM((HALF, BPW), jnp.float32),
            pltpu.VMEM((HALF, D), jnp.float32),
            pltpu.SemaphoreType.DMA,
        ],
    )(_sc_body)
    feat_a, pref_w, wout_w = run(
        node_ids.astype(jnp.int32), features,
        in_sample.astype(jnp.int32), out_sample.astype(jnp.int32),
        jnp.reshape(in_sample_amnt, (n, S)),
        jnp.reshape(out_sample_amnt, (n, S)))

    pref_t = jnp.reshape(jnp.transpose(pref_w, (1, 0, 2)), (RPB, B))
    wout_t = jnp.reshape(jnp.transpose(wout_w, (1, 0, 2)), (RPB, B))

    feat_t = pl.pallas_call(
        _tc_body,
        grid=(RPB // 8, B // D),
        in_specs=[pl.BlockSpec((D, 8, D), lambda r, b: (b, r, 0)),
                  pl.BlockSpec((8, D), lambda r, b: (r, b))],
        out_specs=pl.BlockSpec((D + 1, 8, D), lambda r, b: (0, r, b)),
        out_shape=jax.ShapeDtypeStruct((D + 1, RPB, B), jnp.float32),
    )(feat_a, pref_t)

    feat = jnp.transpose(feat_t, (2, 1, 0))
    weight = jnp.transpose(wout_t, (1, 0))
    return feat, weight


# speedup vs baseline: 1.6073x; 1.0040x over previous
import functools

import jax
import jax.numpy as jnp
from jax import lax
from jax.experimental import pallas as pl
from jax.experimental.pallas import tpu as pltpu
from jax.experimental.pallas import tpu_sc as plsc

NC, NS, L = 2, 16, 16
NW = NC * NS
B, S, D = 256, 16, 128
BPW = B // NW
HALF = S * (2 * S + 2)
RPB = 2 * HALF
NCH = 17


def _sc_body(nid_hbm, feat_hbm, sin_hbm, sout_hbm, win_hbm, wout_hbm,
             fa_hbm, pf_hbm, wt_hbm,
             nid_v, n1in, n1out, w1in, w1out, n2in, n2out, w2in, w2out,
             idx_v, pref8, wout8, outbuf, sem):
    wid = lax.axis_index("s") * NC + lax.axis_index("c")
    base_b = wid * BPW
    pltpu.sync_copy(nid_hbm.at[pl.ds(base_b, BPW)], nid_v)
    l1 = [pltpu.async_copy(sin_hbm.at[nid_v], n1in, sem),
          pltpu.async_copy(sout_hbm.at[nid_v], n1out, sem),
          pltpu.async_copy(win_hbm.at[nid_v], w1in, sem),
          pltpu.async_copy(wout_hbm.at[nid_v], w1out, sem)]
    for d in l1:
        d.wait()

    iota = lax.iota(jnp.int32, L)
    zeros = jnp.zeros((L,), jnp.int32)

    for d1 in range(2):
        n1ref = n1in if d1 == 0 else n1out
        w1ref = w1in if d1 == 0 else w1out
        roff = d1 * HALF

        @pl.loop(0, BPW)
        def _half(i):
            ivec = zeros + i
            l2 = [pltpu.async_copy(sin_hbm.at[n1ref.at[i]], n2in, sem),
                  pltpu.async_copy(sout_hbm.at[n1ref.at[i]], n2out, sem),
                  pltpu.async_copy(win_hbm.at[n1ref.at[i]], w2in, sem),
                  pltpu.async_copy(wout_hbm.at[n1ref.at[i]], w2out, sem)]
            n1row = plsc.load_gather(n1ref, [ivec, iota])
            w1row = plsc.load_gather(w1ref, [ivec, iota])
            pos0 = iota * 34
            pos17 = pos0 + 17
            plsc.store_scatter(idx_v, [pos0 >> 5, pos0 & 31], n1row)
            plsc.store_scatter(idx_v, [pos17 >> 5, pos17 & 31], n1row)
            plsc.store_scatter(pref8, [pos0, ivec], w1row)
            plsc.store_scatter(pref8, [pos17, ivec], w1row)
            w1sq = w1row * w1row
            plsc.store_scatter(wout8, [pos0, ivec], w1sq)
            plsc.store_scatter(wout8, [pos17, ivec], w1sq)
            for d in l2:
                d.wait()
            for j in range(S):
                pin = j * 34 + 1 + iota
                pout = j * 34 + 18 + iota
                plsc.store_scatter(idx_v, [pin >> 5, pin & 31], n2in[j, :])
                plsc.store_scatter(idx_v, [pout >> 5, pout & 31], n2out[j, :])
            fg = [pltpu.async_copy(feat_hbm.at[idx_v.at[c]],
                                   outbuf.at[pl.ds(c * 32, 32), :], sem)
                  for c in range(NCH)]
            for j in range(S):
                w1j = plsc.load_gather(
                    w1ref, [ivec, jnp.full((L,), j, jnp.int32)])
                pin = j * 34 + 1 + iota
                pout = j * 34 + 18 + iota
                w2i = w2in[j, :]
                w2o = w2out[j, :]
                plsc.store_scatter(pref8, [pin, ivec], w2i)
                plsc.store_scatter(pref8, [pout, ivec], w2o)
                plsc.store_scatter(wout8, [pin, ivec], w2i * w1j)
                plsc.store_scatter(wout8, [pout, ivec], w2o * w1j)
            outs = []
            for lo, hi in ((0, 5), (5, 9), (9, 13), (13, 17)):
                for d in fg[lo:hi]:
                    d.wait()
                outs.append(pltpu.async_copy(
                    outbuf.at[pl.ds(lo * 32, (hi - lo) * 32), :],
                    fa_hbm.at[base_b + i,
                              pl.ds(roff + lo * 32, (hi - lo) * 32), :],
                    sem))
            for o in outs:
                o.wait()

        pltpu.sync_copy(pref8, pf_hbm.at[wid, pl.ds(roff, HALF), :])
        pltpu.sync_copy(wout8, wt_hbm.at[wid, pl.ds(roff, HALF), :])


def _tc_body(a_ref, b_ref, o_ref):
    o_ref[0, :, :] = b_ref[...]
    for r in range(8):
        o_ref[pl.ds(1, D), r, :] = jnp.transpose(a_ref[:, r, :], (1, 0))


def kernel(node_ids, features, in_sample, out_sample, in_sample_amnt,
           out_sample_amnt):
    n = features.shape[0]
    mesh = plsc.VectorSubcoreMesh(core_axis_name="c", subcore_axis_name="s")
    run = functools.partial(
        pl.kernel,
        out_type=(jax.ShapeDtypeStruct((B, RPB, D), jnp.float32),
                  jax.ShapeDtypeStruct((NW, RPB, BPW), jnp.float32),
                  jax.ShapeDtypeStruct((NW, RPB, BPW), jnp.float32)),
        mesh=mesh,
        compiler_params=pltpu.CompilerParams(
            use_tc_tiling_on_sc=False, needs_layout_passes=False),
        scratch_types=[
            pltpu.VMEM((BPW,), jnp.int32),
            pltpu.VMEM((BPW, S), jnp.int32),
            pltpu.VMEM((BPW, S), jnp.int32),
            pltpu.VMEM((BPW, S), jnp.float32),
            pltpu.VMEM((BPW, S), jnp.float32),
            pltpu.VMEM((S, S), jnp.int32),
            pltpu.VMEM((S, S), jnp.int32),
            pltpu.VMEM((S, S), jnp.float32),
            pltpu.VMEM((S, S), jnp.float32),
            pltpu.VMEM((NCH, 32), jnp.int32),
            pltpu.VMEM((HALF, BPW), jnp.float32),
            pltpu.VMEM((HALF, BPW), jnp.float32),
            pltpu.VMEM((HALF, D), jnp.float32),
            pltpu.SemaphoreType.DMA,
        ],
    )(_sc_body)
    feat_a, pref_w, wout_w = run(
        node_ids.astype(jnp.int32), features,
        in_sample.astype(jnp.int32), out_sample.astype(jnp.int32),
        jnp.reshape(in_sample_amnt, (n, S)),
        jnp.reshape(out_sample_amnt, (n, S)))

    pref_t = jnp.reshape(jnp.transpose(pref_w, (1, 0, 2)), (RPB, B))
    wout_t = jnp.reshape(jnp.transpose(wout_w, (1, 0, 2)), (RPB, B))

    feat_t = pl.pallas_call(
        _tc_body,
        grid=(RPB // 8, B // D),
        in_specs=[pl.BlockSpec((D, 8, D), lambda r, b: (b, r, 0)),
                  pl.BlockSpec((8, D), lambda r, b: (r, b))],
        out_specs=pl.BlockSpec((D + 1, 8, D), lambda r, b: (0, r, b)),
        out_shape=jax.ShapeDtypeStruct((D + 1, RPB, B), jnp.float32),
    )(feat_a, pref_t)

    feat = jnp.transpose(feat_t, (2, 1, 0))
    weight = jnp.transpose(wout_t, (1, 0))
    return feat, weight
